# baseline (speedup 1.0000x reference)
import functools

import jax
import jax.numpy as jnp
from jax import lax
from jax.experimental import pallas as pl
from jax.experimental.pallas import tpu as pltpu
from jax.experimental.pallas import tpu_sc as plsc

_D_MODEL = 1024
_D_FF = 4096
_E = 8
_K = 2
_M = 2048
_F_TILE = 512
_GB = 256


def _gating_body(x_ref, gw_ref, idx_ref, prob_ref):
    x = x_ref[...]
    gw = gw_ref[...]
    logits = lax.dot_general(x, gw, (((1,), (1,)), ((), ())),
                             preferred_element_type=jnp.float32)
    m = jnp.max(logits, axis=-1, keepdims=True)
    ex = jnp.exp(logits - m)
    p = ex / jnp.sum(ex, axis=-1, keepdims=True)
    lanes = lax.broadcasted_iota(jnp.int32, p.shape, 1)
    big = jnp.int32(_E)
    p1 = jnp.max(p, axis=-1, keepdims=True)
    i1 = jnp.min(jnp.where(p == p1, lanes, big), axis=-1, keepdims=True)
    not1 = lanes != i1
    pm = jnp.where(not1, p, -jnp.inf)
    p2 = jnp.max(pm, axis=-1, keepdims=True)
    i2 = jnp.min(jnp.where((pm == p2) & not1, lanes, big), axis=-1,
                 keepdims=True)
    idx_ref[...] = jnp.concatenate([i1, i2], axis=1)
    prob_ref[...] = jnp.concatenate([p1, p2], axis=1)


def _gating(x_flat, gate_w):
    t = x_flat.shape[0]
    grid = (t // _GB,)
    return pl.pallas_call(
        _gating_body,
        grid=grid,
        in_specs=[
            pl.BlockSpec((_GB, _D_MODEL), lambda i: (i, 0)),
            pl.BlockSpec((_E, _D_MODEL), lambda i: (0, 0)),
        ],
        out_specs=[
            pl.BlockSpec((_GB, _K), lambda i: (i, 0)),
            pl.BlockSpec((_GB, _K), lambda i: (i, 0)),
        ],
        out_shape=[
            jax.ShapeDtypeStruct((t, _K), jnp.int32),
            jax.ShapeDtypeStruct((t, _K), jnp.float32),
        ],
    )(x_flat, gate_w)


def _make_sc_dispatch(d, n_tok, n_slots, chunk):
    info = plsc.get_sparse_core_info()
    nc, ns = info.num_cores, info.num_subcores
    nw = nc * ns
    per_w = n_tok // nw
    assert n_tok % (nw * chunk) == 0
    nchunk = per_w // chunk
    mesh = plsc.VectorSubcoreMesh(core_axis_name="c", subcore_axis_name="s")

    @functools.partial(
        pl.kernel,
        mesh=mesh,
        out_type=jax.ShapeDtypeStruct((n_slots, d), jnp.float32),
        scratch_types=[
            pltpu.VMEM((chunk,), jnp.int32),
            pltpu.VMEM((chunk,), jnp.int32),
            pltpu.VMEM((chunk, d), jnp.float32),
            pltpu.SemaphoreType.DMA,
        ],
    )
    def dispatch_k(x_hbm, de_hbm, do_hbm, out_hbm, ide_v, ido_v, rows_v,
                   sem):
        wid = lax.axis_index("s") * nc + lax.axis_index("c")
        base = wid * per_w

        def body(ci, carry):
            off = base + ci * chunk
            pltpu.sync_copy(x_hbm.at[pl.ds(off, chunk)], rows_v)
            pltpu.sync_copy(de_hbm.at[pl.ds(off, chunk)], ide_v)
            pltpu.sync_copy(do_hbm.at[pl.ds(off, chunk)], ido_v)
            pltpu.async_copy(rows_v, out_hbm.at[ide_v], sem).wait()
            pltpu.async_copy(rows_v, out_hbm.at[ido_v], sem).wait()
            return carry

        lax.fori_loop(0, nchunk, body, 0)

    return dispatch_k


def _make_sc_gather(d, n_idx, chunk):
    info = plsc.get_sparse_core_info()
    nc, ns = info.num_cores, info.num_subcores
    nw = nc * ns
    per_w = n_idx // nw
    assert n_idx % (nw * chunk) == 0 and per_w % chunk == 0
    nchunk = per_w // chunk
    mesh = plsc.VectorSubcoreMesh(core_axis_name="c", subcore_axis_name="s")

    @functools.partial(
        pl.kernel,
        mesh=mesh,
        out_type=jax.ShapeDtypeStruct((n_idx, d), jnp.float32),
        scratch_types=[
            pltpu.VMEM((chunk,), jnp.int32),
            pltpu.VMEM((chunk, d), jnp.float32),
            pltpu.SemaphoreType.DMA,
        ],
    )
    def gather_k(table_hbm, idx_hbm, out_hbm, idx_v, rows_v, sem):
        wid = lax.axis_index("s") * nc + lax.axis_index("c")
        base = wid * per_w

        def body(ci, carry):
            off = base + ci * chunk
            pltpu.sync_copy(idx_hbm.at[pl.ds(off, chunk)], idx_v)
            pltpu.async_copy(table_hbm.at[idx_v], rows_v, sem).wait()
            pltpu.sync_copy(rows_v, out_hbm.at[pl.ds(off, chunk)])
            return carry

        lax.fori_loop(0, nchunk, body, 0)

    return gather_k


def _ffn_body(gid_ref, act_ref, x_ref, w1_ref, b1_ref, w2_ref, b2_ref,
              out_ref):
    i = pl.program_id(0)
    j = pl.program_id(1)

    @pl.when(act_ref[i] == 1)
    def _():
        x = x_ref[...].astype(jnp.bfloat16)
        w1 = w1_ref[0].astype(jnp.bfloat16)
        h = lax.dot_general(x, w1, (((1,), (1,)), ((), ())),
                            preferred_element_type=jnp.float32)
        h = jnp.maximum(h + b1_ref[0], 0.0)
        w2 = w2_ref[0].astype(jnp.bfloat16)
        part = lax.dot_general(h.astype(jnp.bfloat16), w2,
                               (((1,), (1,)), ((), ())),
                               preferred_element_type=jnp.float32)

        @pl.when(j == 0)
        def _():
            out_ref[...] = part + b2_ref[0]

        @pl.when(j > 0)
        def _():
            out_ref[...] += part


def _ffn(x_sorted, w1, b1, w2, b2, gids, active, n_blocks):
    n_ff_tiles = _D_FF // _F_TILE
    grid_spec = pltpu.PrefetchScalarGridSpec(
        num_scalar_prefetch=2,
        grid=(n_blocks, n_ff_tiles),
        in_specs=[
            pl.BlockSpec((_M, _D_MODEL), lambda i, j, g, a: (i, 0)),
            pl.BlockSpec((1, _F_TILE, _D_MODEL),
                         lambda i, j, g, a: (g[i], j, 0)),
            pl.BlockSpec((1, 1, _F_TILE), lambda i, j, g, a: (g[i], 0, j)),
            pl.BlockSpec((1, _D_MODEL, _F_TILE),
                         lambda i, j, g, a: (g[i], 0, j)),
            pl.BlockSpec((1, 1, _D_MODEL), lambda i, j, g, a: (g[i], 0, 0)),
        ],
        out_specs=pl.BlockSpec((_M, _D_MODEL), lambda i, j, g, a: (i, 0)),
    )
    return pl.pallas_call(
        _ffn_body,
        grid_spec=grid_spec,
        out_shape=jax.ShapeDtypeStruct((n_blocks * _M, _D_MODEL),
                                       jnp.float32),
        compiler_params=pltpu.CompilerParams(
            dimension_semantics=("arbitrary", "arbitrary")),
    )(gids, active, x_sorted, w1, b1.reshape(_E, 1, _D_FF), w2,
      b2.reshape(_E, 1, _D_MODEL))


def _combine_body(yg_ref, prob_ref, out_ref):
    gb2, d = yg_ref.shape
    z = yg_ref[...].reshape(gb2 // 2, 2, d)
    y0 = z[:, 0, :]
    y1 = z[:, 1, :]
    p0 = prob_ref[:, 0:1]
    p1 = prob_ref[:, 1:2]
    res = y0 * p0 + y1 * p1
    sr, b, _ = out_ref.shape
    out_ref[...] = res.reshape(sr, b, d)


def _combine(yg, probs, s, b):
    t = probs.shape[0]
    grid = (t // _GB,)
    return pl.pallas_call(
        _combine_body,
        grid=grid,
        in_specs=[
            pl.BlockSpec((_K * _GB, _D_MODEL), lambda i: (i, 0)),
            pl.BlockSpec((_GB, _K), lambda i: (i, 0)),
        ],
        out_specs=pl.BlockSpec((_GB // b, b, _D_MODEL), lambda i: (i, 0, 0)),
        out_shape=jax.ShapeDtypeStruct((s, b, _D_MODEL), jnp.float32),
    )(yg, probs)


def _routing(idx, t):
    p = t * _K
    n_blocks = p // _M + _E
    eids = idx.reshape(p)
    onehot = (eids[:, None] == jnp.arange(_E, dtype=jnp.int32)[None, :]
              ).astype(jnp.int32)
    counts = jnp.sum(onehot, axis=0)
    blocks_per_e = (counts + _M - 1) // _M
    blk_start = jnp.concatenate(
        [jnp.zeros((1,), jnp.int32),
         jnp.cumsum(blocks_per_e)]).astype(jnp.int32)
    padded_off = blk_start[:_E] * _M
    rank = jnp.cumsum(onehot, axis=0) - 1
    rank_p = jnp.take_along_axis(rank, eids[:, None], axis=1)[:, 0]
    dest = padded_off[eids] + rank_p
    total_blocks = blk_start[_E]
    bid = jnp.arange(n_blocks, dtype=jnp.int32)
    gids = jnp.searchsorted(blk_start, bid, side="right").astype(
        jnp.int32) - 1
    gids = jnp.clip(gids, 0, _E - 1)
    last_g = jnp.take(gids, jnp.maximum(total_blocks - 1, 0))
    active = (bid < total_blocks).astype(jnp.int32)
    gids = jnp.where(active == 1, gids, last_g)
    return dest, gids, active, n_blocks


def kernel(x, gate_w, w1, b1, w2, b2):
    s, b, h = x.shape
    t = s * b
    x_flat = x.reshape(t, h)

    idx, probs = _gating(x_flat, gate_w)
    dest, gids, active, n_blocks = _routing(idx, t)

    dk = dest.reshape(t, _K)
    x_sorted = _make_sc_dispatch(_D_MODEL, t, n_blocks * _M, 64)(
        x_flat, dk[:, 0], dk[:, 1])
    y_pairs = _ffn(x_sorted, w1, b1, w2, b2, gids, active, n_blocks)
    yg = _make_sc_gather(_D_MODEL, t * _K, 64)(y_pairs, dest)
    return _combine(yg, probs, s, b)


# speedup vs baseline: 1.2998x; 1.2998x over previous
import functools

import jax
import jax.numpy as jnp
from jax import lax
from jax.experimental import pallas as pl
from jax.experimental.pallas import tpu as pltpu
from jax.experimental.pallas import tpu_sc as plsc

_D_MODEL = 1024
_D_FF = 4096
_E = 8
_K = 2
_M = 1024
_F_TILE = 1024
_GB = 256


def _gating_body(x_ref, gw_ref, idx_ref, prob_ref):
    x = x_ref[...]
    gw = gw_ref[...]
    logits = lax.dot_general(x, gw, (((1,), (1,)), ((), ())),
                             preferred_element_type=jnp.float32)
    m = jnp.max(logits, axis=-1, keepdims=True)
    ex = jnp.exp(logits - m)
    p = ex / jnp.sum(ex, axis=-1, keepdims=True)
    lanes = lax.broadcasted_iota(jnp.int32, p.shape, 1)
    big = jnp.int32(_E)
    p1 = jnp.max(p, axis=-1, keepdims=True)
    i1 = jnp.min(jnp.where(p == p1, lanes, big), axis=-1, keepdims=True)
    not1 = lanes != i1
    pm = jnp.where(not1, p, -jnp.inf)
    p2 = jnp.max(pm, axis=-1, keepdims=True)
    i2 = jnp.min(jnp.where((pm == p2) & not1, lanes, big), axis=-1,
                 keepdims=True)
    idx_ref[...] = jnp.concatenate([i1, i2], axis=1)
    prob_ref[...] = jnp.concatenate([p1, p2], axis=1)


def _gating(x_flat, gate_w):
    t = x_flat.shape[0]
    grid = (t // _GB,)
    return pl.pallas_call(
        _gating_body,
        grid=grid,
        in_specs=[
            pl.BlockSpec((_GB, _D_MODEL), lambda i: (i, 0)),
            pl.BlockSpec((_E, _D_MODEL), lambda i: (0, 0)),
        ],
        out_specs=[
            pl.BlockSpec((_GB, _K), lambda i: (i, 0)),
            pl.BlockSpec((_GB, _K), lambda i: (i, 0)),
        ],
        out_shape=[
            jax.ShapeDtypeStruct((t, _K), jnp.int32),
            jax.ShapeDtypeStruct((t, _K), jnp.float32),
        ],
    )(x_flat, gate_w)


def _make_sc_dispatch(d, n_tok, n_slots, chunk):
    info = plsc.get_sparse_core_info()
    nc, ns = info.num_cores, info.num_subcores
    nw = nc * ns
    per_w = n_tok // nw
    assert n_tok % (nw * chunk) == 0
    nchunk = per_w // chunk
    mesh = plsc.VectorSubcoreMesh(core_axis_name="c", subcore_axis_name="s")

    @functools.partial(
        pl.kernel,
        mesh=mesh,
        out_type=jax.ShapeDtypeStruct((n_slots, d), jnp.float32),
        scratch_types=[
            pltpu.VMEM((chunk,), jnp.int32),
            pltpu.VMEM((chunk,), jnp.int32),
            pltpu.VMEM((chunk, d), jnp.float32),
            pltpu.SemaphoreType.DMA,
        ],
    )
    def dispatch_k(x_hbm, de_hbm, do_hbm, out_hbm, ide_v, ido_v, rows_v,
                   sem):
        wid = lax.axis_index("s") * nc + lax.axis_index("c")
        base = wid * per_w

        def body(ci, carry):
            off = base + ci * chunk
            pltpu.sync_copy(x_hbm.at[pl.ds(off, chunk)], rows_v)
            pltpu.sync_copy(de_hbm.at[pl.ds(off, chunk)], ide_v)
            pltpu.sync_copy(do_hbm.at[pl.ds(off, chunk)], ido_v)
            pltpu.async_copy(rows_v, out_hbm.at[ide_v], sem).wait()
            pltpu.async_copy(rows_v, out_hbm.at[ido_v], sem).wait()
            return carry

        lax.fori_loop(0, nchunk, body, 0)

    return dispatch_k


def _make_sc_gather(d, n_idx, chunk):
    info = plsc.get_sparse_core_info()
    nc, ns = info.num_cores, info.num_subcores
    nw = nc * ns
    per_w = n_idx // nw
    assert n_idx % (nw * chunk) == 0 and per_w % chunk == 0
    nchunk = per_w // chunk
    mesh = plsc.VectorSubcoreMesh(core_axis_name="c", subcore_axis_name="s")

    @functools.partial(
        pl.kernel,
        mesh=mesh,
        out_type=jax.ShapeDtypeStruct((n_idx, d), jnp.float32),
        scratch_types=[
            pltpu.VMEM((chunk,), jnp.int32),
            pltpu.VMEM((chunk, d), jnp.float32),
            pltpu.SemaphoreType.DMA,
        ],
    )
    def gather_k(table_hbm, idx_hbm, out_hbm, idx_v, rows_v, sem):
        wid = lax.axis_index("s") * nc + lax.axis_index("c")
        base = wid * per_w

        def body(ci, carry):
            off = base + ci * chunk
            pltpu.sync_copy(idx_hbm.at[pl.ds(off, chunk)], idx_v)
            pltpu.async_copy(table_hbm.at[idx_v], rows_v, sem).wait()
            pltpu.sync_copy(rows_v, out_hbm.at[pl.ds(off, chunk)])
            return carry

        lax.fori_loop(0, nchunk, body, 0)

    return gather_k


def _ffn_body(gid_ref, act_ref, x_ref, w1_ref, b1_ref, w2_ref, b2_ref,
              out_ref):
    i = pl.program_id(0)
    j = pl.program_id(1)

    @pl.when(act_ref[i] == 1)
    def _():
        x = x_ref[...].astype(jnp.bfloat16)
        w1 = w1_ref[0].astype(jnp.bfloat16)
        h = lax.dot_general(x, w1, (((1,), (1,)), ((), ())),
                            preferred_element_type=jnp.float32)
        h = jnp.maximum(h + b1_ref[0], 0.0)
        w2 = w2_ref[0].astype(jnp.bfloat16)
        part = lax.dot_general(h.astype(jnp.bfloat16), w2,
                               (((1,), (1,)), ((), ())),
                               preferred_element_type=jnp.float32)

        @pl.when(j == 0)
        def _():
            out_ref[...] = part + b2_ref[0]

        @pl.when(j > 0)
        def _():
            out_ref[...] += part


def _ffn(x_sorted, w1, b1, w2, b2, gids, active, n_blocks):
    n_ff_tiles = _D_FF // _F_TILE
    grid_spec = pltpu.PrefetchScalarGridSpec(
        num_scalar_prefetch=2,
        grid=(n_blocks, n_ff_tiles),
        in_specs=[
            pl.BlockSpec((_M, _D_MODEL), lambda i, j, g, a: (i, 0)),
            pl.BlockSpec((1, _F_TILE, _D_MODEL),
                         lambda i, j, g, a: (g[i], j, 0)),
            pl.BlockSpec((1, 1, _F_TILE), lambda i, j, g, a: (g[i], 0, j)),
            pl.BlockSpec((1, _D_MODEL, _F_TILE),
                         lambda i, j, g, a: (g[i], 0, j)),
            pl.BlockSpec((1, 1, _D_MODEL), lambda i, j, g, a: (g[i], 0, 0)),
        ],
        out_specs=pl.BlockSpec((_M, _D_MODEL), lambda i, j, g, a: (i, 0)),
    )
    return pl.pallas_call(
        _ffn_body,
        grid_spec=grid_spec,
        out_shape=jax.ShapeDtypeStruct((n_blocks * _M, _D_MODEL),
                                       jnp.float32),
        compiler_params=pltpu.CompilerParams(
            dimension_semantics=("arbitrary", "arbitrary")),
    )(gids, active, x_sorted, w1, b1.reshape(_E, 1, _D_FF), w2,
      b2.reshape(_E, 1, _D_MODEL))


def _combine_body(yg_ref, prob_ref, out_ref):
    gb2, d = yg_ref.shape
    z = yg_ref[...].reshape(gb2 // 2, 2, d)
    y0 = z[:, 0, :]
    y1 = z[:, 1, :]
    p0 = prob_ref[:, 0:1]
    p1 = prob_ref[:, 1:2]
    res = y0 * p0 + y1 * p1
    sr, b, _ = out_ref.shape
    out_ref[...] = res.reshape(sr, b, d)


def _combine(yg, probs, s, b):
    t = probs.shape[0]
    grid = (t // _GB,)
    return pl.pallas_call(
        _combine_body,
        grid=grid,
        in_specs=[
            pl.BlockSpec((_K * _GB, _D_MODEL), lambda i: (i, 0)),
            pl.BlockSpec((_GB, _K), lambda i: (i, 0)),
        ],
        out_specs=pl.BlockSpec((_GB // b, b, _D_MODEL), lambda i: (i, 0, 0)),
        out_shape=jax.ShapeDtypeStruct((s, b, _D_MODEL), jnp.float32),
    )(yg, probs)


def _routing(idx, t):
    p = t * _K
    n_blocks = p // _M + _E
    eids = idx.reshape(p)
    onehot = (eids[:, None] == jnp.arange(_E, dtype=jnp.int32)[None, :]
              ).astype(jnp.int32)
    counts = jnp.sum(onehot, axis=0)
    blocks_per_e = (counts + _M - 1) // _M
    blk_start = jnp.concatenate(
        [jnp.zeros((1,), jnp.int32),
         jnp.cumsum(blocks_per_e)]).astype(jnp.int32)
    padded_off = blk_start[:_E] * _M
    rank = jnp.cumsum(onehot, axis=0) - 1
    rank_p = jnp.take_along_axis(rank, eids[:, None], axis=1)[:, 0]
    dest = padded_off[eids] + rank_p
    total_blocks = blk_start[_E]
    bid = jnp.arange(n_blocks, dtype=jnp.int32)
    gids = jnp.searchsorted(blk_start, bid, side="right").astype(
        jnp.int32) - 1
    gids = jnp.clip(gids, 0, _E - 1)
    last_g = jnp.take(gids, jnp.maximum(total_blocks - 1, 0))
    active = (bid < total_blocks).astype(jnp.int32)
    gids = jnp.where(active == 1, gids, last_g)
    return dest, gids, active, n_blocks


def kernel(x, gate_w, w1, b1, w2, b2):
    s, b, h = x.shape
    t = s * b
    x_flat = x.reshape(t, h)

    idx, probs = _gating(x_flat, gate_w)
    dest, gids, active, n_blocks = _routing(idx, t)

    dk = dest.reshape(t, _K)
    x_sorted = _make_sc_dispatch(_D_MODEL, t, n_blocks * _M, 64)(
        x_flat, dk[:, 0], dk[:, 1])
    y_pairs = _ffn(x_sorted, w1, b1, w2, b2, gids, active, n_blocks)
    yg = _make_sc_gather(_D_MODEL, t * _K, 64)(y_pairs, dest)
    return _combine(yg, probs, s, b)


# speedup vs baseline: 1.3131x; 1.0102x over previous
import functools

import jax
import jax.numpy as jnp
from jax import lax
from jax.experimental import pallas as pl
from jax.experimental.pallas import tpu as pltpu
from jax.experimental.pallas import tpu_sc as plsc

_D_MODEL = 1024
_D_FF = 4096
_E = 8
_K = 2
_M = 1024
_F_TILE = 1024
_GB = 256


def _gating_body(x_ref, gw_ref, idx_ref, prob_ref):
    x = x_ref[...]
    gw = gw_ref[...]
    logits = lax.dot_general(x, gw, (((1,), (1,)), ((), ())),
                             preferred_element_type=jnp.float32)
    m = jnp.max(logits, axis=-1, keepdims=True)
    ex = jnp.exp(logits - m)
    p = ex / jnp.sum(ex, axis=-1, keepdims=True)
    lanes = lax.broadcasted_iota(jnp.int32, p.shape, 1)
    big = jnp.int32(_E)
    p1 = jnp.max(p, axis=-1, keepdims=True)
    i1 = jnp.min(jnp.where(p == p1, lanes, big), axis=-1, keepdims=True)
    not1 = lanes != i1
    pm = jnp.where(not1, p, -jnp.inf)
    p2 = jnp.max(pm, axis=-1, keepdims=True)
    i2 = jnp.min(jnp.where((pm == p2) & not1, lanes, big), axis=-1,
                 keepdims=True)
    idx_ref[...] = jnp.concatenate([i1, i2], axis=1)
    prob_ref[...] = jnp.concatenate([p1, p2], axis=1)


def _gating(x_flat, gate_w):
    t = x_flat.shape[0]
    grid = (t // _GB,)
    return pl.pallas_call(
        _gating_body,
        grid=grid,
        in_specs=[
            pl.BlockSpec((_GB, _D_MODEL), lambda i: (i, 0)),
            pl.BlockSpec((_E, _D_MODEL), lambda i: (0, 0)),
        ],
        out_specs=[
            pl.BlockSpec((_GB, _K), lambda i: (i, 0)),
            pl.BlockSpec((_GB, _K), lambda i: (i, 0)),
        ],
        out_shape=[
            jax.ShapeDtypeStruct((t, _K), jnp.int32),
            jax.ShapeDtypeStruct((t, _K), jnp.float32),
        ],
    )(x_flat, gate_w)


def _make_sc_dispatch(d, n_tok, n_slots, chunk):
    info = plsc.get_sparse_core_info()
    nc, ns = info.num_cores, info.num_subcores
    nw = nc * ns
    per_w = n_tok // nw
    assert n_tok % (nw * chunk) == 0
    nchunk = per_w // chunk
    mesh = plsc.VectorSubcoreMesh(core_axis_name="c", subcore_axis_name="s")

    @functools.partial(
        pl.kernel,
        mesh=mesh,
        out_type=jax.ShapeDtypeStruct((n_slots, d), jnp.float32),
        scratch_types=[
            pltpu.VMEM((chunk,), jnp.int32),
            pltpu.VMEM((chunk,), jnp.int32),
            pltpu.VMEM((chunk, d), jnp.float32),
            pltpu.SemaphoreType.DMA,
        ],
    )
    def dispatch_k(x_hbm, de_hbm, do_hbm, out_hbm, ide_v, ido_v, rows_v,
                   sem):
        wid = lax.axis_index("s") * nc + lax.axis_index("c")
        base = wid * per_w

        def body(ci, carry):
            off = base + ci * chunk
            pltpu.sync_copy(x_hbm.at[pl.ds(off, chunk)], rows_v)
            pltpu.sync_copy(de_hbm.at[pl.ds(off, chunk)], ide_v)
            pltpu.sync_copy(do_hbm.at[pl.ds(off, chunk)], ido_v)
            pltpu.async_copy(rows_v, out_hbm.at[ide_v], sem).wait()
            pltpu.async_copy(rows_v, out_hbm.at[ido_v], sem).wait()
            return carry

        lax.fori_loop(0, nchunk, body, 0)

    return dispatch_k


def _make_sc_gather(d, n_idx, chunk):
    info = plsc.get_sparse_core_info()
    nc, ns = info.num_cores, info.num_subcores
    nw = nc * ns
    per_w = n_idx // nw
    assert n_idx % (nw * chunk) == 0 and per_w % chunk == 0
    nchunk = per_w // chunk
    mesh = plsc.VectorSubcoreMesh(core_axis_name="c", subcore_axis_name="s")

    @functools.partial(
        pl.kernel,
        mesh=mesh,
        out_type=jax.ShapeDtypeStruct((n_idx, d), jnp.float32),
        scratch_types=[
            pltpu.VMEM((chunk,), jnp.int32),
            pltpu.VMEM((chunk, d), jnp.float32),
            pltpu.SemaphoreType.DMA,
        ],
    )
    def gather_k(table_hbm, idx_hbm, out_hbm, idx_v, rows_v, sem):
        wid = lax.axis_index("s") * nc + lax.axis_index("c")
        base = wid * per_w

        def body(ci, carry):
            off = base + ci * chunk
            pltpu.sync_copy(idx_hbm.at[pl.ds(off, chunk)], idx_v)
            pltpu.async_copy(table_hbm.at[idx_v], rows_v, sem).wait()
            pltpu.sync_copy(rows_v, out_hbm.at[pl.ds(off, chunk)])
            return carry

        lax.fori_loop(0, nchunk, body, 0)

    return gather_k


def _ffn_body(gid_ref, act_ref, rb_ref, x_ref, w1_ref, b1_ref, w2_ref,
              b2_ref, out_ref):
    i = pl.program_id(0)
    j = pl.program_id(1)

    @pl.when(act_ref[i] == 1)
    def _():
        x = x_ref[...].astype(jnp.bfloat16)
        w1 = w1_ref[0].astype(jnp.bfloat16)
        h = lax.dot_general(x, w1, (((1,), (1,)), ((), ())),
                            preferred_element_type=jnp.float32)
        h = jnp.maximum(h + b1_ref[0], 0.0)
        w2 = w2_ref[0].astype(jnp.bfloat16)
        part = lax.dot_general(h.astype(jnp.bfloat16), w2,
                               (((1,), (1,)), ((), ())),
                               preferred_element_type=jnp.float32)

        @pl.when(j == 0)
        def _():
            out_ref[...] = part + b2_ref[0]

        @pl.when(j > 0)
        def _():
            out_ref[...] += part


def _ffn(x_sorted, w1, b1, w2, b2, gids, active, rblk, n_blocks):
    n_ff_tiles = _D_FF // _F_TILE
    grid_spec = pltpu.PrefetchScalarGridSpec(
        num_scalar_prefetch=3,
        grid=(n_blocks, n_ff_tiles),
        in_specs=[
            pl.BlockSpec((_M, _D_MODEL), lambda i, j, g, a, r: (r[i], 0)),
            pl.BlockSpec((1, _F_TILE, _D_MODEL),
                         lambda i, j, g, a, r: (g[i], j, 0)),
            pl.BlockSpec((1, 1, _F_TILE),
                         lambda i, j, g, a, r: (g[i], 0, j)),
            pl.BlockSpec((1, _D_MODEL, _F_TILE),
                         lambda i, j, g, a, r: (g[i], 0, j)),
            pl.BlockSpec((1, 1, _D_MODEL),
                         lambda i, j, g, a, r: (g[i], 0, 0)),
        ],
        out_specs=pl.BlockSpec((_M, _D_MODEL),
                               lambda i, j, g, a, r: (i, 0)),
    )
    return pl.pallas_call(
        _ffn_body,
        grid_spec=grid_spec,
        out_shape=jax.ShapeDtypeStruct((n_blocks * _M, _D_MODEL),
                                       jnp.float32),
        compiler_params=pltpu.CompilerParams(
            dimension_semantics=("parallel", "arbitrary")),
    )(gids, active, rblk, x_sorted, w1, b1.reshape(_E, 1, _D_FF), w2,
      b2.reshape(_E, 1, _D_MODEL))


def _combine_body(yg_ref, prob_ref, out_ref):
    gb2, d = yg_ref.shape
    z = yg_ref[...].reshape(gb2 // 2, 2, d)
    y0 = z[:, 0, :]
    y1 = z[:, 1, :]
    p0 = prob_ref[:, 0:1]
    p1 = prob_ref[:, 1:2]
    res = y0 * p0 + y1 * p1
    sr, b, _ = out_ref.shape
    out_ref[...] = res.reshape(sr, b, d)


def _combine(yg, probs, s, b):
    t = probs.shape[0]
    grid = (t // _GB,)
    return pl.pallas_call(
        _combine_body,
        grid=grid,
        in_specs=[
            pl.BlockSpec((_K * _GB, _D_MODEL), lambda i: (i, 0)),
            pl.BlockSpec((_GB, _K), lambda i: (i, 0)),
        ],
        out_specs=pl.BlockSpec((_GB // b, b, _D_MODEL), lambda i: (i, 0, 0)),
        out_shape=jax.ShapeDtypeStruct((s, b, _D_MODEL), jnp.float32),
    )(yg, probs)


def _routing(idx, t):
    p = t * _K
    n_blocks = p // _M + _E
    eids = idx.reshape(p)
    onehot = (eids[:, None] == jnp.arange(_E, dtype=jnp.int32)[None, :]
              ).astype(jnp.int32)
    counts = jnp.sum(onehot, axis=0)
    blocks_per_e = (counts + _M - 1) // _M
    blk_start = jnp.concatenate(
        [jnp.zeros((1,), jnp.int32),
         jnp.cumsum(blocks_per_e)]).astype(jnp.int32)
    padded_off = blk_start[:_E] * _M
    rank = jnp.cumsum(onehot, axis=0) - 1
    rank_p = jnp.take_along_axis(rank, eids[:, None], axis=1)[:, 0]
    dest = padded_off[eids] + rank_p
    total_blocks = blk_start[_E]
    bid = jnp.arange(n_blocks, dtype=jnp.int32)
    gids = jnp.searchsorted(blk_start, bid, side="right").astype(
        jnp.int32) - 1
    gids = jnp.clip(gids, 0, _E - 1)
    last_g = jnp.take(gids, jnp.maximum(total_blocks - 1, 0))
    active = (bid < total_blocks).astype(jnp.int32)
    gids = jnp.where(active == 1, gids, last_g)
    rblk = jnp.where(active == 1, bid, jnp.maximum(total_blocks - 1, 0))
    return dest, gids, active, rblk, n_blocks


def kernel(x, gate_w, w1, b1, w2, b2):
    s, b, h = x.shape
    t = s * b
    x_flat = x.reshape(t, h)

    idx, probs = _gating(x_flat, gate_w)
    dest, gids, active, rblk, n_blocks = _routing(idx, t)

    dk = dest.reshape(t, _K)
    x_sorted = _make_sc_dispatch(_D_MODEL, t, n_blocks * _M, 64)(
        x_flat, dk[:, 0], dk[:, 1])
    y_pairs = _ffn(x_sorted, w1, b1, w2, b2, gids, active, rblk, n_blocks)
    yg = _make_sc_gather(_D_MODEL, t * _K, 64)(y_pairs, dest)
    return _combine(yg, probs, s, b)


# speedup vs baseline: 1.3317x; 1.0142x over previous
import functools

import jax
import jax.numpy as jnp
from jax import lax
from jax.experimental import pallas as pl
from jax.experimental.pallas import tpu as pltpu
from jax.experimental.pallas import tpu_sc as plsc

_D_MODEL = 1024
_D_FF = 4096
_E = 8
_K = 2
_M = 1536
_SUB = 256
_F_TILE = 1024
_GB = 256


def _gating_body(x_ref, gw_ref, idx_ref, prob_ref):
    x = x_ref[...]
    gw = gw_ref[...]
    logits = lax.dot_general(x, gw, (((1,), (1,)), ((), ())),
                             preferred_element_type=jnp.float32)
    m = jnp.max(logits, axis=-1, keepdims=True)
    ex = jnp.exp(logits - m)
    p = ex / jnp.sum(ex, axis=-1, keepdims=True)
    lanes = lax.broadcasted_iota(jnp.int32, p.shape, 1)
    big = jnp.int32(_E)
    p1 = jnp.max(p, axis=-1, keepdims=True)
    i1 = jnp.min(jnp.where(p == p1, lanes, big), axis=-1, keepdims=True)
    not1 = lanes != i1
    pm = jnp.where(not1, p, -jnp.inf)
    p2 = jnp.max(pm, axis=-1, keepdims=True)
    i2 = jnp.min(jnp.where((pm == p2) & not1, lanes, big), axis=-1,
                 keepdims=True)
    idx_ref[...] = jnp.concatenate([i1, i2], axis=1)
    prob_ref[...] = jnp.concatenate([p1, p2], axis=1)


def _gating(x_flat, gate_w):
    t = x_flat.shape[0]
    grid = (t // _GB,)
    return pl.pallas_call(
        _gating_body,
        grid=grid,
        in_specs=[
            pl.BlockSpec((_GB, _D_MODEL), lambda i: (i, 0)),
            pl.BlockSpec((_E, _D_MODEL), lambda i: (0, 0)),
        ],
        out_specs=[
            pl.BlockSpec((_GB, _K), lambda i: (i, 0)),
            pl.BlockSpec((_GB, _K), lambda i: (i, 0)),
        ],
        out_shape=[
            jax.ShapeDtypeStruct((t, _K), jnp.int32),
            jax.ShapeDtypeStruct((t, _K), jnp.float32),
        ],
    )(x_flat, gate_w)


def _make_sc_dispatch(d, n_tok, n_slots, chunk):
    info = plsc.get_sparse_core_info()
    nc, ns = info.num_cores, info.num_subcores
    nw = nc * ns
    per_w = n_tok // nw
    assert n_tok % (nw * chunk) == 0
    nchunk = per_w // chunk
    mesh = plsc.VectorSubcoreMesh(core_axis_name="c", subcore_axis_name="s")

    @functools.partial(
        pl.kernel,
        mesh=mesh,
        out_type=jax.ShapeDtypeStruct((n_slots, d), jnp.float32),
        scratch_types=[
            pltpu.VMEM((chunk,), jnp.int32),
            pltpu.VMEM((chunk,), jnp.int32),
            pltpu.VMEM((chunk, d), jnp.float32),
            pltpu.SemaphoreType.DMA,
        ],
    )
    def dispatch_k(x_hbm, de_hbm, do_hbm, out_hbm, ide_v, ido_v, rows_v,
                   sem):
        wid = lax.axis_index("s") * nc + lax.axis_index("c")
        base = wid * per_w

        def body(ci, carry):
            off = base + ci * chunk
            pltpu.sync_copy(x_hbm.at[pl.ds(off, chunk)], rows_v)
            pltpu.sync_copy(de_hbm.at[pl.ds(off, chunk)], ide_v)
            pltpu.sync_copy(do_hbm.at[pl.ds(off, chunk)], ido_v)
            pltpu.async_copy(rows_v, out_hbm.at[ide_v], sem).wait()
            pltpu.async_copy(rows_v, out_hbm.at[ido_v], sem).wait()
            return carry

        lax.fori_loop(0, nchunk, body, 0)

    return dispatch_k


def _make_sc_gather(d, n_idx, chunk):
    info = plsc.get_sparse_core_info()
    nc, ns = info.num_cores, info.num_subcores
    nw = nc * ns
    per_w = n_idx // nw
    assert n_idx % (nw * chunk) == 0 and per_w % chunk == 0
    nchunk = per_w // chunk
    mesh = plsc.VectorSubcoreMesh(core_axis_name="c", subcore_axis_name="s")

    @functools.partial(
        pl.kernel,
        mesh=mesh,
        out_type=jax.ShapeDtypeStruct((n_idx, d), jnp.float32),
        scratch_types=[
            pltpu.VMEM((chunk,), jnp.int32),
            pltpu.VMEM((chunk, d), jnp.float32),
            pltpu.SemaphoreType.DMA,
        ],
    )
    def gather_k(table_hbm, idx_hbm, out_hbm, idx_v, rows_v, sem):
        wid = lax.axis_index("s") * nc + lax.axis_index("c")
        base = wid * per_w

        def body(ci, carry):
            off = base + ci * chunk
            pltpu.sync_copy(idx_hbm.at[pl.ds(off, chunk)], idx_v)
            pltpu.async_copy(table_hbm.at[idx_v], rows_v, sem).wait()
            pltpu.sync_copy(rows_v, out_hbm.at[pl.ds(off, chunk)])
            return carry

        lax.fori_loop(0, nchunk, body, 0)

    return gather_k


def _ffn_body(gid_ref, cnt_ref, rb_ref, x_ref, w1_ref, b1_ref, w2_ref,
              b2_ref, out_ref):
    i = pl.program_id(0)
    j = pl.program_id(1)
    cnt = cnt_ref[i]

    @pl.when(cnt > 0)
    def _():
        w1 = w1_ref[0].astype(jnp.bfloat16)
        w2 = w2_ref[0].astype(jnp.bfloat16)
        for s in range(_M // _SUB):
            @pl.when(s * _SUB < cnt)
            def _(s=s):
                x = x_ref[pl.ds(s * _SUB, _SUB), :].astype(jnp.bfloat16)
                h = lax.dot_general(x, w1, (((1,), (1,)), ((), ())),
                                    preferred_element_type=jnp.float32)
                h = jnp.maximum(h + b1_ref[0], 0.0)
                part = lax.dot_general(h.astype(jnp.bfloat16), w2,
                                       (((1,), (1,)), ((), ())),
                                       preferred_element_type=jnp.float32)

                @pl.when(j == 0)
                def _():
                    out_ref[pl.ds(s * _SUB, _SUB), :] = part + b2_ref[0]

                @pl.when(j > 0)
                def _():
                    out_ref[pl.ds(s * _SUB, _SUB), :] += part


def _ffn(x_sorted, w1, b1, w2, b2, gids, cnt, rblk, n_blocks):
    n_ff_tiles = _D_FF // _F_TILE
    grid_spec = pltpu.PrefetchScalarGridSpec(
        num_scalar_prefetch=3,
        grid=(n_blocks, n_ff_tiles),
        in_specs=[
            pl.BlockSpec((_M, _D_MODEL), lambda i, j, g, a, r: (r[i], 0)),
            pl.BlockSpec((1, _F_TILE, _D_MODEL),
                         lambda i, j, g, a, r: (g[i], j, 0)),
            pl.BlockSpec((1, 1, _F_TILE),
                         lambda i, j, g, a, r: (g[i], 0, j)),
            pl.BlockSpec((1, _D_MODEL, _F_TILE),
                         lambda i, j, g, a, r: (g[i], 0, j)),
            pl.BlockSpec((1, 1, _D_MODEL),
                         lambda i, j, g, a, r: (g[i], 0, 0)),
        ],
        out_specs=pl.BlockSpec((_M, _D_MODEL),
                               lambda i, j, g, a, r: (i, 0)),
    )
    return pl.pallas_call(
        _ffn_body,
        grid_spec=grid_spec,
        out_shape=jax.ShapeDtypeStruct((n_blocks * _M, _D_MODEL),
                                       jnp.float32),
        compiler_params=pltpu.CompilerParams(
            dimension_semantics=("parallel", "arbitrary")),
    )(gids, cnt, rblk, x_sorted, w1, b1.reshape(_E, 1, _D_FF), w2,
      b2.reshape(_E, 1, _D_MODEL))


def _combine_body(yg_ref, prob_ref, out_ref):
    gb2, d = yg_ref.shape
    z = yg_ref[...].reshape(gb2 // 2, 2, d)
    y0 = z[:, 0, :]
    y1 = z[:, 1, :]
    p0 = prob_ref[:, 0:1]
    p1 = prob_ref[:, 1:2]
    res = y0 * p0 + y1 * p1
    sr, b, _ = out_ref.shape
    out_ref[...] = res.reshape(sr, b, d)


def _combine(yg, probs, s, b):
    t = probs.shape[0]
    grid = (t // _GB,)
    return pl.pallas_call(
        _combine_body,
        grid=grid,
        in_specs=[
            pl.BlockSpec((_K * _GB, _D_MODEL), lambda i: (i, 0)),
            pl.BlockSpec((_GB, _K), lambda i: (i, 0)),
        ],
        out_specs=pl.BlockSpec((_GB // b, b, _D_MODEL), lambda i: (i, 0, 0)),
        out_shape=jax.ShapeDtypeStruct((s, b, _D_MODEL), jnp.float32),
    )(yg, probs)


def _routing(idx, t):
    p = t * _K
    n_blocks = p // _M + _E
    eids = idx.reshape(p)
    onehot = (eids[:, None] == jnp.arange(_E, dtype=jnp.int32)[None, :]
              ).astype(jnp.int32)
    counts = jnp.sum(onehot, axis=0)
    blocks_per_e = (counts + _M - 1) // _M
    blk_start = jnp.concatenate(
        [jnp.zeros((1,), jnp.int32),
         jnp.cumsum(blocks_per_e)]).astype(jnp.int32)
    padded_off = blk_start[:_E] * _M
    rank = jnp.cumsum(onehot, axis=0) - 1
    rank_p = jnp.take_along_axis(rank, eids[:, None], axis=1)[:, 0]
    dest = padded_off[eids] + rank_p
    total_blocks = blk_start[_E]
    bid = jnp.arange(n_blocks, dtype=jnp.int32)
    gids = jnp.searchsorted(blk_start, bid, side="right").astype(
        jnp.int32) - 1
    gids = jnp.clip(gids, 0, _E - 1)
    last_g = jnp.take(gids, jnp.maximum(total_blocks - 1, 0))
    active = (bid < total_blocks).astype(jnp.int32)
    gids = jnp.where(active == 1, gids, last_g)
    rblk = jnp.where(active == 1, bid, jnp.maximum(total_blocks - 1, 0))
    cnt = jnp.clip(padded_off[gids] + counts[gids] - bid * _M, 0, _M)
    cnt = jnp.where(active == 1, cnt, 0).astype(jnp.int32)
    return dest, gids, cnt, rblk, n_blocks


def kernel(x, gate_w, w1, b1, w2, b2):
    s, b, h = x.shape
    t = s * b
    x_flat = x.reshape(t, h)

    idx, probs = _gating(x_flat, gate_w)
    dest, gids, cnt, rblk, n_blocks = _routing(idx, t)

    dk = dest.reshape(t, _K)
    x_sorted = _make_sc_dispatch(_D_MODEL, t, n_blocks * _M, 64)(
        x_flat, dk[:, 0], dk[:, 1])
    y_pairs = _ffn(x_sorted, w1, b1, w2, b2, gids, cnt, rblk, n_blocks)
    yg = _make_sc_gather(_D_MODEL, t * _K, 64)(y_pairs, dest)
    return _combine(yg, probs, s, b)
